# Initial kernel scaffold; baseline (speedup 1.0000x reference)
import jax
import jax.numpy as jnp
from jax.experimental import pallas as pl


def kernel(t, xh, node_mask, edge_mask, params):
    raise NotImplementedError("write your pallas kernel here")


# speedup vs baseline: 6.9092x; 6.9092x over previous
import jax
import jax.numpy as jnp
from jax.experimental import pallas as pl
from jax.experimental.pallas import tpu as pltpu

BS = 16
N = 64
H = 128
IN_NF = 9
CI = 8
NCH = N // CI
INV_NORM = 0.01


def _silu(v):
    return v * jax.nn.sigmoid(v)


def _dot(a, b):
    return jnp.dot(a, b, preferred_element_type=jnp.float32)


def _nt(a, b):
    return jax.lax.dot_general(a, b, (((1,), (1,)), ((), ())),
                               preferred_element_type=jnp.float32)


def _pairwise(x, xc):
    g = _nt(x, x)
    eye = (jax.lax.broadcasted_iota(jnp.int32, (N, N), 0) ==
           jax.lax.broadcasted_iota(jnp.int32, (N, N), 1)).astype(jnp.float32)
    rrow = jnp.sum(g * eye, axis=0, keepdims=True)
    gc = _nt(xc, x)
    rcol = jnp.sum(xc * xc, axis=1, keepdims=True)
    return jnp.maximum(rcol + rrow - 2.0 * gc, 0.0)


def _embed_kernel(hin_ref, w0_ref, b0_ref, w1_ref, b1_ref, out_ref):
    h = _silu(_dot(hin_ref[0], w0_ref[:]) + b0_ref[:])
    out_ref[0] = _dot(h, w1_ref[:]) + b1_ref[:]


def _edge_pre(hc, hf, x, x0, xc, x0c, wr, wc, wd, wt, b1):
    d2 = _pairwise(x, xc)
    dist = _pairwise(x0, x0c)
    a = _dot(hc, wr)
    b = _dot(hf, wc)
    m1 = (a[:, None, :] + b[None, :, :]
          + d2[:, :, None] * wd[None, :, :]
          + dist[:, :, None] * wt[None, :, :]
          + b1[None, :, :])
    return _silu(m1), d2


def _gcl_kernel(hf_ref, hc_ref, x_ref, x0_ref, xc_ref, x0c_ref,
                em_ref, nm_ref,
                wr_ref, wc_ref, wd_ref, wt_ref, b1_ref, w2_ref, b2_ref,
                nwh_ref, nwa_ref, nb1_ref, nw2_ref, nb2_ref, out_ref):
    hf = hf_ref[0]
    hc = hc_ref[0]
    m1, _ = _edge_pre(hc, hf, x_ref[0], x0_ref[0], xc_ref[0], x0c_ref[0],
                      wr_ref[:], wc_ref[:], wd_ref[:], wt_ref[:], b1_ref[:])
    m2 = _silu(_dot(m1.reshape(CI * N, H), w2_ref[:]) + b2_ref[:])
    m3 = m2.reshape(CI, N, H) * em_ref[0][:, :, None]
    agg = jnp.sum(m3, axis=1) * INV_NORM
    u = _silu(_dot(hc, nwh_ref[:]) + _dot(agg, nwa_ref[:]) + nb1_ref[:])
    u = _dot(u, nw2_ref[:]) + nb2_ref[:]
    out_ref[0] = (hc + u) * nm_ref[0]


def _coord_kernel(hf_ref, hc_ref, x_ref, x0_ref, xc_ref, x0c_ref, em_ref,
                  wr_ref, wc_ref, wd_ref, wt_ref, b1_ref, w2_ref, b2_ref,
                  w3_ref, out_ref):
    hf = hf_ref[0]
    hc = hc_ref[0]
    xc = xc_ref[0]
    p1, d2 = _edge_pre(hc, hf, x_ref[0], x0_ref[0], xc, x0c_ref[0],
                       wr_ref[:], wc_ref[:], wd_ref[:], wt_ref[:],
                       b1_ref[:])
    p2 = _silu(_dot(p1.reshape(CI * N, H), w2_ref[:]) + b2_ref[:])
    s = jnp.sum(p2.reshape(CI, N, H) * w3_ref[:][None, :, :], axis=2)
    norm = jnp.sqrt(d2 + 1e-8)
    w = s * em_ref[0] * (INV_NORM / 1.0) / (norm + 1.0)
    delta = xc * jnp.sum(w, axis=1, keepdims=True) - _dot(w, x_ref[0])
    out_ref[0] = xc + delta


def _out_kernel(h_ref, x_ref, x0_ref, nm_ref,
                w0_ref, b0_ref, w1_ref, b1_ref, w2_ref, b2_ref,
                vel_ref, hf_ref):
    nm = nm_ref[0]
    h = _silu(_dot(h_ref[0], w0_ref[:]) + b0_ref[:])
    h = _silu(_dot(h, w1_ref[:]) + b1_ref[:])
    hf_ref[0] = (_dot(h, w2_ref[:]) + b2_ref[:]) * nm
    vel_ref[0] = (x_ref[0] - x0_ref[0]) * nm


def _w_specs(shapes, grid_rank):
    if grid_rank == 1:
        return [pl.BlockSpec(s, lambda b: (0,) * len(s)) for s in shapes]
    return [pl.BlockSpec(s, lambda b, c: (0,) * len(s)) for s in shapes]


_PAR = pltpu.CompilerParams(dimension_semantics=("parallel", "parallel"))
_PAR1 = pltpu.CompilerParams(dimension_semantics=("parallel",))


def _split_first(p):
    w = p["w"]
    return (w[:H], w[H:2 * H], w[2 * H:2 * H + 1], w[2 * H + 1:2 * H + 2],
            p["b"].reshape(1, H))


def kernel(t, xh, node_mask, edge_mask, params):
    nm = node_mask
    xhm = xh * nm
    x0 = xhm[:, :, :3]
    ht = jnp.broadcast_to(t[:, None, :], (BS, N, 1))
    hin = jnp.concatenate([xhm[:, :, 3:], ht], axis=-1)
    em = edge_mask.reshape(BS, N, N)

    emb = params["embedding"]
    h = pl.pallas_call(
        _embed_kernel,
        grid=(BS,),
        in_specs=[pl.BlockSpec((1, N, IN_NF), lambda b: (b, 0, 0))]
        + _w_specs([(IN_NF, H), (1, H), (H, H), (1, H)], 1),
        out_specs=pl.BlockSpec((1, N, H), lambda b: (b, 0, 0)),
        out_shape=jax.ShapeDtypeStruct((BS, N, H), jnp.float32),
        compiler_params=_PAR1,
    )(hin, emb[0]["w"], emb[0]["b"].reshape(1, H),
      emb[1]["w"], emb[1]["b"].reshape(1, H))

    x = x0
    for blk in params["blocks"]:
        for gcl in blk["gcls"]:
            ewr, ewc, ewd, ewt, eb1 = _split_first(gcl["edge_mlp"][0])
            ew2 = gcl["edge_mlp"][1]["w"]
            eb2 = gcl["edge_mlp"][1]["b"].reshape(1, H)
            nw = gcl["node_mlp"][0]["w"]
            nwh, nwa = nw[:H], nw[H:]
            nb1 = gcl["node_mlp"][0]["b"].reshape(1, H)
            nw2 = gcl["node_mlp"][1]["w"]
            nb2 = gcl["node_mlp"][1]["b"].reshape(1, H)
            h = pl.pallas_call(
                _gcl_kernel,
                grid=(BS, NCH),
                in_specs=[
                    pl.BlockSpec((1, N, H), lambda b, c: (b, 0, 0)),
                    pl.BlockSpec((1, CI, H), lambda b, c: (b, c, 0)),
                    pl.BlockSpec((1, N, 3), lambda b, c: (b, 0, 0)),
                    pl.BlockSpec((1, N, 3), lambda b, c: (b, 0, 0)),
                    pl.BlockSpec((1, CI, 3), lambda b, c: (b, c, 0)),
                    pl.BlockSpec((1, CI, 3), lambda b, c: (b, c, 0)),
                    pl.BlockSpec((1, CI, N), lambda b, c: (b, c, 0)),
                    pl.BlockSpec((1, CI, 1), lambda b, c: (b, c, 0)),
                ] + _w_specs([(H, H), (H, H), (1, H), (1, H), (1, H),
                              (H, H), (1, H), (H, H), (H, H), (1, H),
                              (H, H), (1, H)], 2),
                out_specs=pl.BlockSpec((1, CI, H), lambda b, c: (b, c, 0)),
                out_shape=jax.ShapeDtypeStruct((BS, N, H), jnp.float32),
                compiler_params=_PAR,
            )(h, h, x, x0, x, x0, em, nm,
              ewr, ewc, ewd, ewt, eb1, ew2, eb2,
              nwh, nwa, nb1, nw2, nb2)

        cwr, cwc, cwd, cwt, cb1 = _split_first(blk["coord_mlp"][0])
        cw2 = blk["coord_mlp"][1]["w"]
        cb2 = blk["coord_mlp"][1]["b"].reshape(1, H)
        cw3 = blk["coord_mlp"][2]["w"].reshape(1, H)
        x = pl.pallas_call(
            _coord_kernel,
            grid=(BS, NCH),
            in_specs=[
                pl.BlockSpec((1, N, H), lambda b, c: (b, 0, 0)),
                pl.BlockSpec((1, CI, H), lambda b, c: (b, c, 0)),
                pl.BlockSpec((1, N, 3), lambda b, c: (b, 0, 0)),
                pl.BlockSpec((1, N, 3), lambda b, c: (b, 0, 0)),
                pl.BlockSpec((1, CI, 3), lambda b, c: (b, c, 0)),
                pl.BlockSpec((1, CI, 3), lambda b, c: (b, c, 0)),
                pl.BlockSpec((1, CI, N), lambda b, c: (b, c, 0)),
            ] + _w_specs([(H, H), (H, H), (1, H), (1, H), (1, H),
                          (H, H), (1, H), (1, H)], 2),
            out_specs=pl.BlockSpec((1, CI, 3), lambda b, c: (b, c, 0)),
            out_shape=jax.ShapeDtypeStruct((BS, N, 3), jnp.float32),
            compiler_params=_PAR,
        )(h, h, x, x0, x, x0, em, cwr, cwc, cwd, cwt, cb1, cw2, cb2, cw3)

    eo = params["embedding_out"]
    vel, hf = pl.pallas_call(
        _out_kernel,
        grid=(BS,),
        in_specs=[
            pl.BlockSpec((1, N, H), lambda b: (b, 0, 0)),
            pl.BlockSpec((1, N, 3), lambda b: (b, 0, 0)),
            pl.BlockSpec((1, N, 3), lambda b: (b, 0, 0)),
            pl.BlockSpec((1, N, 1), lambda b: (b, 0, 0)),
        ] + _w_specs([(H, H), (1, H), (H, H), (1, H), (H, IN_NF),
                      (1, IN_NF)], 1),
        out_specs=[pl.BlockSpec((1, N, 3), lambda b: (b, 0, 0)),
                   pl.BlockSpec((1, N, IN_NF), lambda b: (b, 0, 0))],
        out_shape=[jax.ShapeDtypeStruct((BS, N, 3), jnp.float32),
                   jax.ShapeDtypeStruct((BS, N, IN_NF), jnp.float32)],
        compiler_params=_PAR1,
    )(h, x, x0, nm,
      eo[0]["w"], eo[0]["b"].reshape(1, H),
      eo[1]["w"], eo[1]["b"].reshape(1, H),
      eo[2]["w"], eo[2]["b"].reshape(1, IN_NF))

    return jnp.concatenate([vel, hf[:, :, :8]], axis=-1)


# speedup vs baseline: 8.2525x; 1.1944x over previous
import jax
import jax.numpy as jnp
from jax.experimental import pallas as pl
from jax.experimental.pallas import tpu as pltpu

BS = 16
N = 64
H = 128
IN_NF = 9
CI = 8
NCH = N // CI
N_LAYERS = 4
SUB = 2


def _silu(v):
    return v * jax.nn.sigmoid(v)


def _dot(a, b):
    return jnp.dot(a, b, preferred_element_type=jnp.float32)


def _nt(a, b):
    return jax.lax.dot_general(a, b, (((1,), (1,)), ((), ())),
                               preferred_element_type=jnp.float32)


def _pairwise_full(x):
    g = _nt(x, x)
    eye = (jax.lax.broadcasted_iota(jnp.int32, (N, N), 0) ==
           jax.lax.broadcasted_iota(jnp.int32, (N, N), 1)).astype(jnp.float32)
    rrow = jnp.sum(g * eye, axis=0, keepdims=True)
    rcol = jnp.sum(g * eye, axis=1, keepdims=True)
    return jnp.maximum(rcol + rrow - 2.0 * g, 0.0)


def _fused_kernel(hin_ref, x0_ref, em_ref, nm_ref,
                  ew0_ref, eb0_ref, ew1_ref, eb1e_ref,
                  gwr_ref, gwc_ref, gwd_ref, gwt_ref, gb1_ref,
                  gw2_ref, gb2_ref,
                  nwh_ref, nwa_ref, nb1_ref, nw2_ref, nb2_ref,
                  cwr_ref, cwc_ref, cwd_ref, cwt_ref, cb1_ref,
                  cw2_ref, cb2_ref, cw3_ref,
                  ow0_ref, ob0_ref, ow1_ref, ob1_ref, ow2_ref, ob2_ref,
                  vel_ref, hf_ref,
                  ha_ref, hb_ref, xa_ref, xb_ref, d2_ref, dist_ref):
    nm = nm_ref[0]
    x0 = x0_ref[0]

    h = _silu(_dot(hin_ref[0], ew0_ref[:]) + eb0_ref[:])
    ha_ref[:] = _dot(h, ew1_ref[:]) + eb1e_ref[:]

    dist_ref[:] = _pairwise_full(x0)

    h_refs = (ha_ref, hb_ref)
    x_refs = (xa_ref, xb_ref)
    h_cur = 0
    x_cur = 0
    xa_ref[:] = x0

    for layer in range(N_LAYERS):
        xr = x_refs[x_cur]
        d2_ref[:] = _pairwise_full(xr[:])

        for sub in range(SUB):
            k = layer * SUB + sub
            hr = h_refs[h_cur]
            hn = h_refs[1 - h_cur]
            hf = hr[:]
            b = _dot(hf, gwc_ref[k])
            wr = gwr_ref[k]
            wd = gwd_ref[k]
            wt = gwt_ref[k]
            b1 = gb1_ref[k]
            w2 = gw2_ref[k]
            b2 = gb2_ref[k]
            nwh = nwh_ref[k]
            nwa = nwa_ref[k]
            nb1 = nb1_ref[k]
            nw2 = nw2_ref[k]
            nb2 = nb2_ref[k]

            def gcl_chunk(c, _, hf=hf, b=b, hr=hr, hn=hn, wr=wr, wd=wd,
                          wt=wt, b1=b1, w2=w2, b2=b2, nwh=nwh, nwa=nwa,
                          nb1=nb1, nw2=nw2, nb2=nb2):
                sl = pl.ds(c * CI, CI)
                hc = hr[sl, :]
                d2 = d2_ref[sl, :]
                dist = dist_ref[sl, :]
                a = _dot(hc, wr) + b1
                m1 = _silu(a[:, None, :] + b[None, :, :]
                           + d2[:, :, None] * wd[None, :, :]
                           + dist[:, :, None] * wt[None, :, :])
                m2 = _silu(_dot(m1.reshape(CI * N, H), w2) + b2)
                m3 = m2.reshape(CI, N, H) * em_ref[0, sl, :][:, :, None]
                agg = jnp.sum(m3, axis=1)
                u = _silu(_dot(hc, nwh) + _dot(agg, nwa) + nb1)
                u = _dot(u, nw2) + nb2
                hn[sl, :] = (hc + u) * nm_ref[0, sl, :]
                return 0

            jax.lax.fori_loop(0, NCH, gcl_chunk, 0)
            h_cur = 1 - h_cur

        hr = h_refs[h_cur]
        hf = hr[:]
        b = _dot(hf, cwc_ref[layer])
        xn = x_refs[1 - x_cur]
        wr = cwr_ref[layer]
        wd = cwd_ref[layer]
        wt = cwt_ref[layer]
        b1 = cb1_ref[layer]
        w2 = cw2_ref[layer]
        b2 = cb2_ref[layer]
        w3 = cw3_ref[layer]

        def coord_chunk(c, _, hf=hf, b=b, hr=hr, xr=xr, xn=xn, wr=wr,
                        wd=wd, wt=wt, b1=b1, w2=w2, b2=b2, w3=w3):
            sl = pl.ds(c * CI, CI)
            hc = hr[sl, :]
            xc = xr[sl, :]
            d2 = d2_ref[sl, :]
            dist = dist_ref[sl, :]
            a = _dot(hc, wr) + b1
            p1 = _silu(a[:, None, :] + b[None, :, :]
                       + d2[:, :, None] * wd[None, :, :]
                       + dist[:, :, None] * wt[None, :, :])
            p2 = _silu(_dot(p1.reshape(CI * N, H), w2) + b2)
            s = jnp.sum(p2.reshape(CI, N, H) * w3[None, :, :], axis=2)
            norm = jnp.sqrt(d2 + 1e-8)
            w = s * em_ref[0, sl, :] / (norm + 1.0)
            delta = xc * jnp.sum(w, axis=1, keepdims=True) - _dot(w, xr[:])
            xn[sl, :] = xc + delta
            return 0

        jax.lax.fori_loop(0, NCH, coord_chunk, 0)
        x_cur = 1 - x_cur

    h = h_refs[h_cur][:]
    h = _silu(_dot(h, ow0_ref[:]) + ob0_ref[:])
    h = _silu(_dot(h, ow1_ref[:]) + ob1_ref[:])
    hf_ref[0] = (_dot(h, ow2_ref[:]) + ob2_ref[:]) * nm
    vel_ref[0] = (x_refs[x_cur][:] - x0) * nm


def _stack(blocks, get):
    return jnp.stack([get(b) for b in blocks])


def kernel(t, xh, node_mask, edge_mask, params):
    nm = node_mask
    xhm = xh * nm
    x0 = xhm[:, :, :3]
    ht = jnp.broadcast_to(t[:, None, :], (BS, N, 1))
    hin = jnp.concatenate([xhm[:, :, 3:], ht], axis=-1)
    em = edge_mask.reshape(BS, N, N)

    gcls = [g for blk in params["blocks"] for g in blk["gcls"]]
    coords = [blk["coord_mlp"] for blk in params["blocks"]]

    def first_splits(layers, idx):
        ws = [l[idx]["w"] for l in layers]
        return (jnp.stack([w[:H] for w in ws]),
                jnp.stack([w[H:2 * H] for w in ws]),
                jnp.stack([w[2 * H:2 * H + 1] for w in ws]),
                jnp.stack([w[2 * H + 1:2 * H + 2] for w in ws]),
                jnp.stack([l[idx]["b"].reshape(1, H) for l in layers]))

    edge_mlps = [g["edge_mlp"] for g in gcls]
    gwr, gwc, gwd, gwt, gb1 = first_splits(edge_mlps, 0)
    gw2 = jnp.stack([e[1]["w"] for e in edge_mlps])
    gb2 = jnp.stack([e[1]["b"].reshape(1, H) for e in edge_mlps])
    nwh = jnp.stack([g["node_mlp"][0]["w"][:H] for g in gcls])
    nwa = jnp.stack([g["node_mlp"][0]["w"][H:] * 0.01 for g in gcls])
    nb1 = jnp.stack([g["node_mlp"][0]["b"].reshape(1, H) for g in gcls])
    nw2 = jnp.stack([g["node_mlp"][1]["w"] for g in gcls])
    nb2 = jnp.stack([g["node_mlp"][1]["b"].reshape(1, H) for g in gcls])

    gwr2, gwc2, gwd2, gwt2, cb1 = first_splits(coords, 0)
    cw2 = jnp.stack([c[1]["w"] for c in coords])
    cb2 = jnp.stack([c[1]["b"].reshape(1, H) for c in coords])
    cw3 = jnp.stack([c[2]["w"].reshape(1, H) * 0.01 for c in coords])

    emb = params["embedding"]
    eo = params["embedding_out"]

    full = lambda s: pl.BlockSpec(s, lambda b, _s=len(s): (0,) * _s)
    per_mol = lambda s: pl.BlockSpec(s, lambda b: (b,) + (0,) * (len(s) - 1))

    vel, hf = pl.pallas_call(
        _fused_kernel,
        grid=(BS,),
        in_specs=[
            per_mol((1, N, IN_NF)), per_mol((1, N, 3)),
            per_mol((1, N, N)), per_mol((1, N, 1)),
            full((IN_NF, H)), full((1, H)), full((H, H)), full((1, H)),
            full((SUB * N_LAYERS, H, H)), full((SUB * N_LAYERS, H, H)),
            full((SUB * N_LAYERS, 1, H)), full((SUB * N_LAYERS, 1, H)),
            full((SUB * N_LAYERS, 1, H)), full((SUB * N_LAYERS, H, H)),
            full((SUB * N_LAYERS, 1, H)),
            full((SUB * N_LAYERS, H, H)), full((SUB * N_LAYERS, H, H)),
            full((SUB * N_LAYERS, 1, H)), full((SUB * N_LAYERS, H, H)),
            full((SUB * N_LAYERS, 1, H)),
            full((N_LAYERS, H, H)), full((N_LAYERS, H, H)),
            full((N_LAYERS, 1, H)), full((N_LAYERS, 1, H)),
            full((N_LAYERS, 1, H)), full((N_LAYERS, H, H)),
            full((N_LAYERS, 1, H)), full((N_LAYERS, 1, H)),
            full((H, H)), full((1, H)), full((H, H)), full((1, H)),
            full((H, IN_NF)), full((1, IN_NF)),
        ],
        out_specs=[per_mol((1, N, 3)), per_mol((1, N, IN_NF))],
        out_shape=[jax.ShapeDtypeStruct((BS, N, 3), jnp.float32),
                   jax.ShapeDtypeStruct((BS, N, IN_NF), jnp.float32)],
        scratch_shapes=[
            pltpu.VMEM((N, H), jnp.float32), pltpu.VMEM((N, H), jnp.float32),
            pltpu.VMEM((N, 3), jnp.float32), pltpu.VMEM((N, 3), jnp.float32),
            pltpu.VMEM((N, N), jnp.float32), pltpu.VMEM((N, N), jnp.float32),
        ],
        compiler_params=pltpu.CompilerParams(
            dimension_semantics=("parallel",)),
    )(hin, x0, em, nm,
      emb[0]["w"], emb[0]["b"].reshape(1, H),
      emb[1]["w"], emb[1]["b"].reshape(1, H),
      gwr, gwc, gwd, gwt, gb1, gw2, gb2,
      nwh, nwa, nb1, nw2, nb2,
      gwr2, gwc2, gwd2, gwt2, cb1, cw2, cb2, cw3,
      eo[0]["w"], eo[0]["b"].reshape(1, H),
      eo[1]["w"], eo[1]["b"].reshape(1, H),
      eo[2]["w"], eo[2]["b"].reshape(1, IN_NF))

    return jnp.concatenate([vel, hf[:, :, :8]], axis=-1)


# speedup vs baseline: 9.1589x; 1.1098x over previous
import jax
import jax.numpy as jnp
from jax.experimental import pallas as pl
from jax.experimental.pallas import tpu as pltpu

BS = 16
N = 64
H = 128
IN_NF = 9
CI = 8
NCH = N // CI
N_LAYERS = 4
SUB = 2


def _silu(v):
    return v * jax.nn.sigmoid(v)


def _dot(a, b):
    return jnp.dot(a, b, preferred_element_type=jnp.float32)


def _nt(a, b):
    return jax.lax.dot_general(a, b, (((1,), (1,)), ((), ())),
                               preferred_element_type=jnp.float32)


def _pairwise_full(x):
    g = _nt(x, x)
    eye = (jax.lax.broadcasted_iota(jnp.int32, (N, N), 0) ==
           jax.lax.broadcasted_iota(jnp.int32, (N, N), 1)).astype(jnp.float32)
    rrow = jnp.sum(g * eye, axis=0, keepdims=True)
    rcol = jnp.sum(g * eye, axis=1, keepdims=True)
    return jnp.maximum(rcol + rrow - 2.0 * g, 0.0)


def _fused_kernel(hin_ref, x0_ref, em_ref, nm_ref,
                  ew0_ref, eb0_ref, ew1_ref, eb1e_ref,
                  gwr_ref, gwc_ref, gwd_ref, gwt_ref, gb1_ref,
                  gw2_ref, gb2_ref,
                  nwh_ref, nwa_ref, nb1_ref, nw2_ref, nb2_ref,
                  cwr_ref, cwc_ref, cwd_ref, cwt_ref, cb1_ref,
                  cw2_ref, cb2_ref, cw3_ref,
                  ow0_ref, ob0_ref, ow1_ref, ob1_ref, ow2_ref, ob2_ref,
                  vel_ref, hf_ref,
                  ha_ref, hb_ref, xa_ref, xb_ref, d2_ref, dist_ref):
    nm = nm_ref[0]
    x0 = x0_ref[0]

    h = _silu(_dot(hin_ref[0], ew0_ref[:]) + eb0_ref[:])
    ha_ref[:] = _dot(h, ew1_ref[:]) + eb1e_ref[:]

    dist_ref[:] = _pairwise_full(x0)

    h_refs = (ha_ref, hb_ref)
    x_refs = (xa_ref, xb_ref)
    h_cur = 0
    x_cur = 0
    xa_ref[:] = x0

    for layer in range(N_LAYERS):
        xr = x_refs[x_cur]
        d2_ref[:] = _pairwise_full(xr[:])

        for sub in range(SUB):
            k = layer * SUB + sub
            hr = h_refs[h_cur]
            hn = h_refs[1 - h_cur]
            hf = hr[:]
            b = _dot(hf, gwc_ref[k])
            wr = gwr_ref[k]
            wd = gwd_ref[k]
            wt = gwt_ref[k]
            b1 = gb1_ref[k]
            w2 = gw2_ref[k]
            b2 = gb2_ref[k]
            nwh = nwh_ref[k]
            nwa = nwa_ref[k]
            nb1 = nb1_ref[k]
            nw2 = nw2_ref[k]
            nb2 = nb2_ref[k]

            def gcl_chunk(c, _, hf=hf, b=b, hr=hr, hn=hn, wr=wr, wd=wd,
                          wt=wt, b1=b1, w2=w2, b2=b2, nwh=nwh, nwa=nwa,
                          nb1=nb1, nw2=nw2, nb2=nb2):
                sl = pl.ds(c * CI, CI)
                hc = hr[sl, :]
                d2 = d2_ref[sl, :]
                dist = dist_ref[sl, :]
                a = _dot(hc, wr) + b1
                m1 = _silu(a[:, None, :] + b[None, :, :]
                           + d2[:, :, None] * wd[None, :, :]
                           + dist[:, :, None] * wt[None, :, :])
                m2 = _silu(_dot(m1.reshape(CI * N, H), w2) + b2)
                m3 = m2.reshape(CI, N, H) * em_ref[0, sl, :][:, :, None]
                agg = jnp.sum(m3, axis=1)
                u = _silu(_dot(hc, nwh) + _dot(agg, nwa) + nb1)
                u = _dot(u, nw2) + nb2
                hn[sl, :] = (hc + u) * nm_ref[0, sl, :]
                return 0

            jax.lax.fori_loop(0, NCH, gcl_chunk, 0, unroll=2)
            h_cur = 1 - h_cur

        hr = h_refs[h_cur]
        hf = hr[:]
        b = _dot(hf, cwc_ref[layer])
        xn = x_refs[1 - x_cur]
        wr = cwr_ref[layer]
        wd = cwd_ref[layer]
        wt = cwt_ref[layer]
        b1 = cb1_ref[layer]
        w2 = cw2_ref[layer]
        b2 = cb2_ref[layer]
        w3 = cw3_ref[layer]

        def coord_chunk(c, _, hf=hf, b=b, hr=hr, xr=xr, xn=xn, wr=wr,
                        wd=wd, wt=wt, b1=b1, w2=w2, b2=b2, w3=w3):
            sl = pl.ds(c * CI, CI)
            hc = hr[sl, :]
            xc = xr[sl, :]
            d2 = d2_ref[sl, :]
            dist = dist_ref[sl, :]
            a = _dot(hc, wr) + b1
            p1 = _silu(a[:, None, :] + b[None, :, :]
                       + d2[:, :, None] * wd[None, :, :]
                       + dist[:, :, None] * wt[None, :, :])
            p2 = _silu(_dot(p1.reshape(CI * N, H), w2) + b2)
            s = jnp.sum(p2.reshape(CI, N, H) * w3[None, :, :], axis=2)
            norm = jnp.sqrt(d2 + 1e-8)
            w = s * em_ref[0, sl, :] / (norm + 1.0)
            delta = xc * jnp.sum(w, axis=1, keepdims=True) - _dot(w, xr[:])
            xn[sl, :] = xc + delta
            return 0

        jax.lax.fori_loop(0, NCH, coord_chunk, 0, unroll=2)
        x_cur = 1 - x_cur

    h = h_refs[h_cur][:]
    h = _silu(_dot(h, ow0_ref[:]) + ob0_ref[:])
    h = _silu(_dot(h, ow1_ref[:]) + ob1_ref[:])
    hf_ref[0] = (_dot(h, ow2_ref[:]) + ob2_ref[:]) * nm
    vel_ref[0] = (x_refs[x_cur][:] - x0) * nm


def _stack(blocks, get):
    return jnp.stack([get(b) for b in blocks])


def kernel(t, xh, node_mask, edge_mask, params):
    nm = node_mask
    xhm = xh * nm
    x0 = xhm[:, :, :3]
    ht = jnp.broadcast_to(t[:, None, :], (BS, N, 1))
    hin = jnp.concatenate([xhm[:, :, 3:], ht], axis=-1)
    em = edge_mask.reshape(BS, N, N)

    gcls = [g for blk in params["blocks"] for g in blk["gcls"]]
    coords = [blk["coord_mlp"] for blk in params["blocks"]]

    def first_splits(layers, idx):
        ws = [l[idx]["w"] for l in layers]
        return (jnp.stack([w[:H] for w in ws]),
                jnp.stack([w[H:2 * H] for w in ws]),
                jnp.stack([w[2 * H:2 * H + 1] for w in ws]),
                jnp.stack([w[2 * H + 1:2 * H + 2] for w in ws]),
                jnp.stack([l[idx]["b"].reshape(1, H) for l in layers]))

    edge_mlps = [g["edge_mlp"] for g in gcls]
    gwr, gwc, gwd, gwt, gb1 = first_splits(edge_mlps, 0)
    gw2 = jnp.stack([e[1]["w"] for e in edge_mlps])
    gb2 = jnp.stack([e[1]["b"].reshape(1, H) for e in edge_mlps])
    nwh = jnp.stack([g["node_mlp"][0]["w"][:H] for g in gcls])
    nwa = jnp.stack([g["node_mlp"][0]["w"][H:] * 0.01 for g in gcls])
    nb1 = jnp.stack([g["node_mlp"][0]["b"].reshape(1, H) for g in gcls])
    nw2 = jnp.stack([g["node_mlp"][1]["w"] for g in gcls])
    nb2 = jnp.stack([g["node_mlp"][1]["b"].reshape(1, H) for g in gcls])

    gwr2, gwc2, gwd2, gwt2, cb1 = first_splits(coords, 0)
    cw2 = jnp.stack([c[1]["w"] for c in coords])
    cb2 = jnp.stack([c[1]["b"].reshape(1, H) for c in coords])
    cw3 = jnp.stack([c[2]["w"].reshape(1, H) * 0.01 for c in coords])

    emb = params["embedding"]
    eo = params["embedding_out"]

    full = lambda s: pl.BlockSpec(s, lambda b, _s=len(s): (0,) * _s)
    per_mol = lambda s: pl.BlockSpec(s, lambda b: (b,) + (0,) * (len(s) - 1))

    vel, hf = pl.pallas_call(
        _fused_kernel,
        grid=(BS,),
        in_specs=[
            per_mol((1, N, IN_NF)), per_mol((1, N, 3)),
            per_mol((1, N, N)), per_mol((1, N, 1)),
            full((IN_NF, H)), full((1, H)), full((H, H)), full((1, H)),
            full((SUB * N_LAYERS, H, H)), full((SUB * N_LAYERS, H, H)),
            full((SUB * N_LAYERS, 1, H)), full((SUB * N_LAYERS, 1, H)),
            full((SUB * N_LAYERS, 1, H)), full((SUB * N_LAYERS, H, H)),
            full((SUB * N_LAYERS, 1, H)),
            full((SUB * N_LAYERS, H, H)), full((SUB * N_LAYERS, H, H)),
            full((SUB * N_LAYERS, 1, H)), full((SUB * N_LAYERS, H, H)),
            full((SUB * N_LAYERS, 1, H)),
            full((N_LAYERS, H, H)), full((N_LAYERS, H, H)),
            full((N_LAYERS, 1, H)), full((N_LAYERS, 1, H)),
            full((N_LAYERS, 1, H)), full((N_LAYERS, H, H)),
            full((N_LAYERS, 1, H)), full((N_LAYERS, 1, H)),
            full((H, H)), full((1, H)), full((H, H)), full((1, H)),
            full((H, IN_NF)), full((1, IN_NF)),
        ],
        out_specs=[per_mol((1, N, 3)), per_mol((1, N, IN_NF))],
        out_shape=[jax.ShapeDtypeStruct((BS, N, 3), jnp.float32),
                   jax.ShapeDtypeStruct((BS, N, IN_NF), jnp.float32)],
        scratch_shapes=[
            pltpu.VMEM((N, H), jnp.float32), pltpu.VMEM((N, H), jnp.float32),
            pltpu.VMEM((N, 3), jnp.float32), pltpu.VMEM((N, 3), jnp.float32),
            pltpu.VMEM((N, N), jnp.float32), pltpu.VMEM((N, N), jnp.float32),
        ],
        compiler_params=pltpu.CompilerParams(
            dimension_semantics=("parallel",)),
    )(hin, x0, em, nm,
      emb[0]["w"], emb[0]["b"].reshape(1, H),
      emb[1]["w"], emb[1]["b"].reshape(1, H),
      gwr, gwc, gwd, gwt, gb1, gw2, gb2,
      nwh, nwa, nb1, nw2, nb2,
      gwr2, gwc2, gwd2, gwt2, cb1, cw2, cb2, cw3,
      eo[0]["w"], eo[0]["b"].reshape(1, H),
      eo[1]["w"], eo[1]["b"].reshape(1, H),
      eo[2]["w"], eo[2]["b"].reshape(1, IN_NF))

    return jnp.concatenate([vel, hf[:, :, :8]], axis=-1)


# speedup vs baseline: 11.6618x; 1.2733x over previous
import jax
import jax.numpy as jnp
from jax.experimental import pallas as pl
from jax.experimental.pallas import tpu as pltpu

BS = 16
N = 64
H = 128
IN_NF = 9
CI = 16
NCH = N // CI
N_LAYERS = 4
SUB = 2


def _silu(v):
    return v * jax.nn.sigmoid(v)


def _dot(a, b):
    return jnp.dot(a, b, preferred_element_type=jnp.float32)


def _nt(a, b):
    return jax.lax.dot_general(a, b, (((1,), (1,)), ((), ())),
                               preferred_element_type=jnp.float32)


def _pairwise_full(x):
    g = _nt(x, x)
    eye = (jax.lax.broadcasted_iota(jnp.int32, (N, N), 0) ==
           jax.lax.broadcasted_iota(jnp.int32, (N, N), 1)).astype(jnp.float32)
    rrow = jnp.sum(g * eye, axis=0, keepdims=True)
    rcol = jnp.sum(g * eye, axis=1, keepdims=True)
    return jnp.maximum(rcol + rrow - 2.0 * g, 0.0)


def _fused_kernel(hin_ref, x0_ref, em_ref, nm_ref,
                  ew0_ref, eb0_ref, ew1_ref, eb1e_ref,
                  gwr_ref, gwc_ref, gwd_ref, gwt_ref, gb1_ref,
                  gw2_ref, gb2_ref,
                  nwh_ref, nwa_ref, nb1_ref, nw2_ref, nb2_ref,
                  cwr_ref, cwc_ref, cwd_ref, cwt_ref, cb1_ref,
                  cw2_ref, cb2_ref, cw3_ref,
                  ow0_ref, ob0_ref, ow1_ref, ob1_ref, ow2_ref, ob2_ref,
                  vel_ref, hf_ref,
                  ha_ref, hb_ref, xa_ref, xb_ref, d2_ref, dist_ref):
    nm = nm_ref[0]
    x0 = x0_ref[0]

    h = _silu(_dot(hin_ref[0], ew0_ref[:]) + eb0_ref[:])
    ha_ref[:] = _dot(h, ew1_ref[:]) + eb1e_ref[:]

    dist_ref[:] = _pairwise_full(x0)

    h_refs = (ha_ref, hb_ref)
    x_refs = (xa_ref, xb_ref)
    h_cur = 0
    x_cur = 0
    xa_ref[:] = x0

    for layer in range(N_LAYERS):
        xr = x_refs[x_cur]
        d2_ref[:] = _pairwise_full(xr[:])

        for sub in range(SUB):
            k = layer * SUB + sub
            hr = h_refs[h_cur]
            hn = h_refs[1 - h_cur]
            hf = hr[:]
            b = _dot(hf, gwc_ref[k])
            wr = gwr_ref[k]
            wd = gwd_ref[k]
            wt = gwt_ref[k]
            b1 = gb1_ref[k]
            w2 = gw2_ref[k]
            b2 = gb2_ref[k]
            nwh = nwh_ref[k]
            nwa = nwa_ref[k]
            nb1 = nb1_ref[k]
            nw2 = nw2_ref[k]
            nb2 = nb2_ref[k]

            def gcl_chunk(c, _, hf=hf, b=b, hr=hr, hn=hn, wr=wr, wd=wd,
                          wt=wt, b1=b1, w2=w2, b2=b2, nwh=nwh, nwa=nwa,
                          nb1=nb1, nw2=nw2, nb2=nb2):
                sl = pl.ds(c * CI, CI)
                hc = hr[sl, :]
                d2 = d2_ref[sl, :]
                dist = dist_ref[sl, :]
                a = _dot(hc, wr) + b1
                m1 = _silu(a[:, None, :] + b[None, :, :]
                           + d2[:, :, None] * wd[None, :, :]
                           + dist[:, :, None] * wt[None, :, :])
                m2 = _silu(_dot(m1.reshape(CI * N, H), w2) + b2)
                m3 = m2.reshape(CI, N, H) * em_ref[0, sl, :][:, :, None]
                agg = jnp.sum(m3, axis=1)
                u = _silu(_dot(hc, nwh) + _dot(agg, nwa) + nb1)
                u = _dot(u, nw2) + nb2
                hn[sl, :] = (hc + u) * nm_ref[0, sl, :]
                return 0

            jax.lax.fori_loop(0, NCH, gcl_chunk, 0)
            h_cur = 1 - h_cur

        hr = h_refs[h_cur]
        hf = hr[:]
        b = _dot(hf, cwc_ref[layer])
        xn = x_refs[1 - x_cur]
        wr = cwr_ref[layer]
        wd = cwd_ref[layer]
        wt = cwt_ref[layer]
        b1 = cb1_ref[layer]
        w2 = cw2_ref[layer]
        b2 = cb2_ref[layer]
        w3 = cw3_ref[layer]

        def coord_chunk(c, _, hf=hf, b=b, hr=hr, xr=xr, xn=xn, wr=wr,
                        wd=wd, wt=wt, b1=b1, w2=w2, b2=b2, w3=w3):
            sl = pl.ds(c * CI, CI)
            hc = hr[sl, :]
            xc = xr[sl, :]
            d2 = d2_ref[sl, :]
            dist = dist_ref[sl, :]
            a = _dot(hc, wr) + b1
            p1 = _silu(a[:, None, :] + b[None, :, :]
                       + d2[:, :, None] * wd[None, :, :]
                       + dist[:, :, None] * wt[None, :, :])
            p2 = _silu(_dot(p1.reshape(CI * N, H), w2) + b2)
            s = jnp.sum(p2.reshape(CI, N, H) * w3[None, :, :], axis=2)
            norm = jnp.sqrt(d2 + 1e-8)
            w = s * em_ref[0, sl, :] / (norm + 1.0)
            delta = xc * jnp.sum(w, axis=1, keepdims=True) - _dot(w, xr[:])
            xn[sl, :] = xc + delta
            return 0

        jax.lax.fori_loop(0, NCH, coord_chunk, 0)
        x_cur = 1 - x_cur

    h = h_refs[h_cur][:]
    h = _silu(_dot(h, ow0_ref[:]) + ob0_ref[:])
    h = _silu(_dot(h, ow1_ref[:]) + ob1_ref[:])
    hf_ref[0] = (_dot(h, ow2_ref[:]) + ob2_ref[:]) * nm
    vel_ref[0] = (x_refs[x_cur][:] - x0) * nm


def _stack(blocks, get):
    return jnp.stack([get(b) for b in blocks])


def kernel(t, xh, node_mask, edge_mask, params):
    nm = node_mask
    xhm = xh * nm
    x0 = xhm[:, :, :3]
    ht = jnp.broadcast_to(t[:, None, :], (BS, N, 1))
    hin = jnp.concatenate([xhm[:, :, 3:], ht], axis=-1)
    em = edge_mask.reshape(BS, N, N)

    gcls = [g for blk in params["blocks"] for g in blk["gcls"]]
    coords = [blk["coord_mlp"] for blk in params["blocks"]]

    def first_splits(layers, idx):
        ws = [l[idx]["w"] for l in layers]
        return (jnp.stack([w[:H] for w in ws]),
                jnp.stack([w[H:2 * H] for w in ws]),
                jnp.stack([w[2 * H:2 * H + 1] for w in ws]),
                jnp.stack([w[2 * H + 1:2 * H + 2] for w in ws]),
                jnp.stack([l[idx]["b"].reshape(1, H) for l in layers]))

    edge_mlps = [g["edge_mlp"] for g in gcls]
    gwr, gwc, gwd, gwt, gb1 = first_splits(edge_mlps, 0)
    gw2 = jnp.stack([e[1]["w"] for e in edge_mlps])
    gb2 = jnp.stack([e[1]["b"].reshape(1, H) for e in edge_mlps])
    nwh = jnp.stack([g["node_mlp"][0]["w"][:H] for g in gcls])
    nwa = jnp.stack([g["node_mlp"][0]["w"][H:] * 0.01 for g in gcls])
    nb1 = jnp.stack([g["node_mlp"][0]["b"].reshape(1, H) for g in gcls])
    nw2 = jnp.stack([g["node_mlp"][1]["w"] for g in gcls])
    nb2 = jnp.stack([g["node_mlp"][1]["b"].reshape(1, H) for g in gcls])

    gwr2, gwc2, gwd2, gwt2, cb1 = first_splits(coords, 0)
    cw2 = jnp.stack([c[1]["w"] for c in coords])
    cb2 = jnp.stack([c[1]["b"].reshape(1, H) for c in coords])
    cw3 = jnp.stack([c[2]["w"].reshape(1, H) * 0.01 for c in coords])

    emb = params["embedding"]
    eo = params["embedding_out"]

    full = lambda s: pl.BlockSpec(s, lambda b, _s=len(s): (0,) * _s)
    per_mol = lambda s: pl.BlockSpec(s, lambda b: (b,) + (0,) * (len(s) - 1))

    vel, hf = pl.pallas_call(
        _fused_kernel,
        grid=(BS,),
        in_specs=[
            per_mol((1, N, IN_NF)), per_mol((1, N, 3)),
            per_mol((1, N, N)), per_mol((1, N, 1)),
            full((IN_NF, H)), full((1, H)), full((H, H)), full((1, H)),
            full((SUB * N_LAYERS, H, H)), full((SUB * N_LAYERS, H, H)),
            full((SUB * N_LAYERS, 1, H)), full((SUB * N_LAYERS, 1, H)),
            full((SUB * N_LAYERS, 1, H)), full((SUB * N_LAYERS, H, H)),
            full((SUB * N_LAYERS, 1, H)),
            full((SUB * N_LAYERS, H, H)), full((SUB * N_LAYERS, H, H)),
            full((SUB * N_LAYERS, 1, H)), full((SUB * N_LAYERS, H, H)),
            full((SUB * N_LAYERS, 1, H)),
            full((N_LAYERS, H, H)), full((N_LAYERS, H, H)),
            full((N_LAYERS, 1, H)), full((N_LAYERS, 1, H)),
            full((N_LAYERS, 1, H)), full((N_LAYERS, H, H)),
            full((N_LAYERS, 1, H)), full((N_LAYERS, 1, H)),
            full((H, H)), full((1, H)), full((H, H)), full((1, H)),
            full((H, IN_NF)), full((1, IN_NF)),
        ],
        out_specs=[per_mol((1, N, 3)), per_mol((1, N, IN_NF))],
        out_shape=[jax.ShapeDtypeStruct((BS, N, 3), jnp.float32),
                   jax.ShapeDtypeStruct((BS, N, IN_NF), jnp.float32)],
        scratch_shapes=[
            pltpu.VMEM((N, H), jnp.float32), pltpu.VMEM((N, H), jnp.float32),
            pltpu.VMEM((N, 3), jnp.float32), pltpu.VMEM((N, 3), jnp.float32),
            pltpu.VMEM((N, N), jnp.float32), pltpu.VMEM((N, N), jnp.float32),
        ],
        compiler_params=pltpu.CompilerParams(
            dimension_semantics=("parallel",)),
    )(hin, x0, em, nm,
      emb[0]["w"], emb[0]["b"].reshape(1, H),
      emb[1]["w"], emb[1]["b"].reshape(1, H),
      gwr, gwc, gwd, gwt, gb1, gw2, gb2,
      nwh, nwa, nb1, nw2, nb2,
      gwr2, gwc2, gwd2, gwt2, cb1, cw2, cb2, cw3,
      eo[0]["w"], eo[0]["b"].reshape(1, H),
      eo[1]["w"], eo[1]["b"].reshape(1, H),
      eo[2]["w"], eo[2]["b"].reshape(1, IN_NF))

    return jnp.concatenate([vel, hf[:, :, :8]], axis=-1)


# speedup vs baseline: 12.5208x; 1.0737x over previous
import jax
import jax.numpy as jnp
from jax.experimental import pallas as pl
from jax.experimental.pallas import tpu as pltpu

BS = 16
N = 64
H = 128
IN_NF = 9
CI = 16
NCH = N // CI
N_LAYERS = 4
SUB = 2


def _silu(v):
    return v * jax.nn.sigmoid(v)


def _dot(a, b):
    return jnp.dot(a, b, preferred_element_type=jnp.float32)


def _nt(a, b):
    return jax.lax.dot_general(a, b, (((1,), (1,)), ((), ())),
                               preferred_element_type=jnp.float32)


def _pairwise_full(x):
    g = _nt(x, x)
    eye = (jax.lax.broadcasted_iota(jnp.int32, (N, N), 0) ==
           jax.lax.broadcasted_iota(jnp.int32, (N, N), 1)).astype(jnp.float32)
    rrow = jnp.sum(g * eye, axis=0, keepdims=True)
    rcol = jnp.sum(g * eye, axis=1, keepdims=True)
    return jnp.maximum(rcol + rrow - 2.0 * g, 0.0)


def _fused_kernel(hin_ref, x0_ref, em_ref, nm_ref,
                  ew0_ref, eb0_ref, ew1_ref, eb1e_ref,
                  gwr_ref, gwc_ref, gwd_ref, gwt_ref, gb1_ref,
                  gw2_ref, gb2_ref,
                  nwh_ref, nwa_ref, nb1_ref, nw2_ref, nb2_ref,
                  cwr_ref, cwc_ref, cwd_ref, cwt_ref, cb1_ref,
                  cw2_ref, cb2_ref, cw3_ref,
                  ow0_ref, ob0_ref, ow1_ref, ob1_ref, ow2_ref, ob2_ref,
                  vel_ref, hf_ref,
                  ha_ref, hb_ref, xa_ref, xb_ref, d2_ref, dist_ref):
    nm = nm_ref[0]
    x0 = x0_ref[0]

    h = _silu(_dot(hin_ref[0], ew0_ref[:]) + eb0_ref[:])
    ha_ref[:] = _dot(h, ew1_ref[:]) + eb1e_ref[:]

    dist_ref[:] = _pairwise_full(x0)

    h_refs = (ha_ref, hb_ref)
    x_refs = (xa_ref, xb_ref)
    h_cur = 0
    x_cur = 0
    xa_ref[:] = x0

    for layer in range(N_LAYERS):
        xr = x_refs[x_cur]
        d2_ref[:] = _pairwise_full(xr[:])

        for sub in range(SUB):
            k = layer * SUB + sub
            hr = h_refs[h_cur]
            hn = h_refs[1 - h_cur]
            hf = hr[:]
            b = _dot(hf, gwc_ref[k])
            wr = gwr_ref[k]
            wd = gwd_ref[k]
            wt = gwt_ref[k]
            b1 = gb1_ref[k]
            w2 = gw2_ref[k]
            b2 = gb2_ref[k]
            nwh = nwh_ref[k]
            nwa = nwa_ref[k]
            nb1 = nb1_ref[k]
            nw2 = nw2_ref[k]
            nb2 = nb2_ref[k]

            def gcl_chunk(c, _, hf=hf, b=b, hr=hr, hn=hn, wr=wr, wd=wd,
                          wt=wt, b1=b1, w2=w2, b2=b2, nwh=nwh, nwa=nwa,
                          nb1=nb1, nw2=nw2, nb2=nb2):
                sl = pl.ds(c * CI, CI)
                hc = hr[sl, :]
                d2 = d2_ref[sl, :]
                dist = dist_ref[sl, :]
                a = _dot(hc, wr) + b1
                m1 = _silu(a[:, None, :] + b[None, :, :]
                           + d2[:, :, None] * wd[None, :, :]
                           + dist[:, :, None] * wt[None, :, :])
                m2 = _silu(_dot(m1.reshape(CI * N, H), w2) + b2)
                m3 = m2.reshape(CI, N, H) * em_ref[0, sl, :][:, :, None]
                agg = jnp.sum(m3, axis=1)
                u = _silu(_dot(hc, nwh) + _dot(agg, nwa) + nb1)
                u = _dot(u, nw2) + nb2
                hn[sl, :] = (hc + u) * nm_ref[0, sl, :]
                return 0

            jax.lax.fori_loop(0, NCH, gcl_chunk, 0, unroll=2)
            h_cur = 1 - h_cur

        hr = h_refs[h_cur]
        hf = hr[:]
        b = _dot(hf, cwc_ref[layer])
        xn = x_refs[1 - x_cur]
        wr = cwr_ref[layer]
        wd = cwd_ref[layer]
        wt = cwt_ref[layer]
        b1 = cb1_ref[layer]
        w2 = cw2_ref[layer]
        b2 = cb2_ref[layer]
        w3 = cw3_ref[layer]

        def coord_chunk(c, _, hf=hf, b=b, hr=hr, xr=xr, xn=xn, wr=wr,
                        wd=wd, wt=wt, b1=b1, w2=w2, b2=b2, w3=w3):
            sl = pl.ds(c * CI, CI)
            hc = hr[sl, :]
            xc = xr[sl, :]
            d2 = d2_ref[sl, :]
            dist = dist_ref[sl, :]
            a = _dot(hc, wr) + b1
            p1 = _silu(a[:, None, :] + b[None, :, :]
                       + d2[:, :, None] * wd[None, :, :]
                       + dist[:, :, None] * wt[None, :, :])
            p2 = _silu(_dot(p1.reshape(CI * N, H), w2) + b2)
            s = jnp.sum(p2.reshape(CI, N, H) * w3[None, :, :], axis=2)
            norm = jnp.sqrt(d2 + 1e-8)
            w = s * em_ref[0, sl, :] / (norm + 1.0)
            delta = xc * jnp.sum(w, axis=1, keepdims=True) - _dot(w, xr[:])
            xn[sl, :] = xc + delta
            return 0

        jax.lax.fori_loop(0, NCH, coord_chunk, 0, unroll=2)
        x_cur = 1 - x_cur

    h = h_refs[h_cur][:]
    h = _silu(_dot(h, ow0_ref[:]) + ob0_ref[:])
    h = _silu(_dot(h, ow1_ref[:]) + ob1_ref[:])
    hf_ref[0] = (_dot(h, ow2_ref[:]) + ob2_ref[:]) * nm
    vel_ref[0] = (x_refs[x_cur][:] - x0) * nm


def _stack(blocks, get):
    return jnp.stack([get(b) for b in blocks])


def kernel(t, xh, node_mask, edge_mask, params):
    nm = node_mask
    xhm = xh * nm
    x0 = xhm[:, :, :3]
    ht = jnp.broadcast_to(t[:, None, :], (BS, N, 1))
    hin = jnp.concatenate([xhm[:, :, 3:], ht], axis=-1)
    em = edge_mask.reshape(BS, N, N)

    gcls = [g for blk in params["blocks"] for g in blk["gcls"]]
    coords = [blk["coord_mlp"] for blk in params["blocks"]]

    def first_splits(layers, idx):
        ws = [l[idx]["w"] for l in layers]
        return (jnp.stack([w[:H] for w in ws]),
                jnp.stack([w[H:2 * H] for w in ws]),
                jnp.stack([w[2 * H:2 * H + 1] for w in ws]),
                jnp.stack([w[2 * H + 1:2 * H + 2] for w in ws]),
                jnp.stack([l[idx]["b"].reshape(1, H) for l in layers]))

    edge_mlps = [g["edge_mlp"] for g in gcls]
    gwr, gwc, gwd, gwt, gb1 = first_splits(edge_mlps, 0)
    gw2 = jnp.stack([e[1]["w"] for e in edge_mlps])
    gb2 = jnp.stack([e[1]["b"].reshape(1, H) for e in edge_mlps])
    nwh = jnp.stack([g["node_mlp"][0]["w"][:H] for g in gcls])
    nwa = jnp.stack([g["node_mlp"][0]["w"][H:] * 0.01 for g in gcls])
    nb1 = jnp.stack([g["node_mlp"][0]["b"].reshape(1, H) for g in gcls])
    nw2 = jnp.stack([g["node_mlp"][1]["w"] for g in gcls])
    nb2 = jnp.stack([g["node_mlp"][1]["b"].reshape(1, H) for g in gcls])

    gwr2, gwc2, gwd2, gwt2, cb1 = first_splits(coords, 0)
    cw2 = jnp.stack([c[1]["w"] for c in coords])
    cb2 = jnp.stack([c[1]["b"].reshape(1, H) for c in coords])
    cw3 = jnp.stack([c[2]["w"].reshape(1, H) * 0.01 for c in coords])

    emb = params["embedding"]
    eo = params["embedding_out"]

    full = lambda s: pl.BlockSpec(s, lambda b, _s=len(s): (0,) * _s)
    per_mol = lambda s: pl.BlockSpec(s, lambda b: (b,) + (0,) * (len(s) - 1))

    vel, hf = pl.pallas_call(
        _fused_kernel,
        grid=(BS,),
        in_specs=[
            per_mol((1, N, IN_NF)), per_mol((1, N, 3)),
            per_mol((1, N, N)), per_mol((1, N, 1)),
            full((IN_NF, H)), full((1, H)), full((H, H)), full((1, H)),
            full((SUB * N_LAYERS, H, H)), full((SUB * N_LAYERS, H, H)),
            full((SUB * N_LAYERS, 1, H)), full((SUB * N_LAYERS, 1, H)),
            full((SUB * N_LAYERS, 1, H)), full((SUB * N_LAYERS, H, H)),
            full((SUB * N_LAYERS, 1, H)),
            full((SUB * N_LAYERS, H, H)), full((SUB * N_LAYERS, H, H)),
            full((SUB * N_LAYERS, 1, H)), full((SUB * N_LAYERS, H, H)),
            full((SUB * N_LAYERS, 1, H)),
            full((N_LAYERS, H, H)), full((N_LAYERS, H, H)),
            full((N_LAYERS, 1, H)), full((N_LAYERS, 1, H)),
            full((N_LAYERS, 1, H)), full((N_LAYERS, H, H)),
            full((N_LAYERS, 1, H)), full((N_LAYERS, 1, H)),
            full((H, H)), full((1, H)), full((H, H)), full((1, H)),
            full((H, IN_NF)), full((1, IN_NF)),
        ],
        out_specs=[per_mol((1, N, 3)), per_mol((1, N, IN_NF))],
        out_shape=[jax.ShapeDtypeStruct((BS, N, 3), jnp.float32),
                   jax.ShapeDtypeStruct((BS, N, IN_NF), jnp.float32)],
        scratch_shapes=[
            pltpu.VMEM((N, H), jnp.float32), pltpu.VMEM((N, H), jnp.float32),
            pltpu.VMEM((N, 3), jnp.float32), pltpu.VMEM((N, 3), jnp.float32),
            pltpu.VMEM((N, N), jnp.float32), pltpu.VMEM((N, N), jnp.float32),
        ],
        compiler_params=pltpu.CompilerParams(
            dimension_semantics=("parallel",)),
    )(hin, x0, em, nm,
      emb[0]["w"], emb[0]["b"].reshape(1, H),
      emb[1]["w"], emb[1]["b"].reshape(1, H),
      gwr, gwc, gwd, gwt, gb1, gw2, gb2,
      nwh, nwa, nb1, nw2, nb2,
      gwr2, gwc2, gwd2, gwt2, cb1, cw2, cb2, cw3,
      eo[0]["w"], eo[0]["b"].reshape(1, H),
      eo[1]["w"], eo[1]["b"].reshape(1, H),
      eo[2]["w"], eo[2]["b"].reshape(1, IN_NF))

    return jnp.concatenate([vel, hf[:, :, :8]], axis=-1)


# speedup vs baseline: 14.0441x; 1.1217x over previous
import jax
import jax.numpy as jnp
from jax.experimental import pallas as pl
from jax.experimental.pallas import tpu as pltpu

BS = 16
N = 64
H = 128
IN_NF = 9
CI = 32
NCH = N // CI
N_LAYERS = 4
SUB = 2


def _silu(v):
    return v * jax.nn.sigmoid(v)


def _dot(a, b):
    return jnp.dot(a, b, preferred_element_type=jnp.float32)


def _nt(a, b):
    return jax.lax.dot_general(a, b, (((1,), (1,)), ((), ())),
                               preferred_element_type=jnp.float32)


def _pairwise_full(x):
    g = _nt(x, x)
    eye = (jax.lax.broadcasted_iota(jnp.int32, (N, N), 0) ==
           jax.lax.broadcasted_iota(jnp.int32, (N, N), 1)).astype(jnp.float32)
    rrow = jnp.sum(g * eye, axis=0, keepdims=True)
    rcol = jnp.sum(g * eye, axis=1, keepdims=True)
    return jnp.maximum(rcol + rrow - 2.0 * g, 0.0)


def _fused_kernel(hin_ref, x0_ref, em_ref, nm_ref,
                  ew0_ref, eb0_ref, ew1_ref, eb1e_ref,
                  gwr_ref, gwc_ref, gwd_ref, gwt_ref, gb1_ref,
                  gw2_ref, gb2_ref,
                  nwh_ref, nwa_ref, nb1_ref, nw2_ref, nb2_ref,
                  cwr_ref, cwc_ref, cwd_ref, cwt_ref, cb1_ref,
                  cw2_ref, cb2_ref, cw3_ref,
                  ow0_ref, ob0_ref, ow1_ref, ob1_ref, ow2_ref, ob2_ref,
                  vel_ref, hf_ref,
                  ha_ref, hb_ref, xa_ref, xb_ref, d2_ref, dist_ref):
    nm = nm_ref[0]
    x0 = x0_ref[0]

    h = _silu(_dot(hin_ref[0], ew0_ref[:]) + eb0_ref[:])
    ha_ref[:] = _dot(h, ew1_ref[:]) + eb1e_ref[:]

    dist_ref[:] = _pairwise_full(x0)

    h_refs = (ha_ref, hb_ref)
    x_refs = (xa_ref, xb_ref)
    h_cur = 0
    x_cur = 0
    xa_ref[:] = x0

    for layer in range(N_LAYERS):
        xr = x_refs[x_cur]
        d2_ref[:] = _pairwise_full(xr[:])

        for sub in range(SUB):
            k = layer * SUB + sub
            hr = h_refs[h_cur]
            hn = h_refs[1 - h_cur]
            hf = hr[:]
            b = _dot(hf, gwc_ref[k])
            wr = gwr_ref[k]
            wd = gwd_ref[k]
            wt = gwt_ref[k]
            b1 = gb1_ref[k]
            w2 = gw2_ref[k]
            b2 = gb2_ref[k]
            nwh = nwh_ref[k]
            nwa = nwa_ref[k]
            nb1 = nb1_ref[k]
            nw2 = nw2_ref[k]
            nb2 = nb2_ref[k]

            def gcl_chunk(c, _, hf=hf, b=b, hr=hr, hn=hn, wr=wr, wd=wd,
                          wt=wt, b1=b1, w2=w2, b2=b2, nwh=nwh, nwa=nwa,
                          nb1=nb1, nw2=nw2, nb2=nb2):
                sl = pl.ds(c * CI, CI)
                hc = hr[sl, :]
                d2 = d2_ref[sl, :]
                dist = dist_ref[sl, :]
                a = _dot(hc, wr) + b1
                m1 = _silu(a[:, None, :] + b[None, :, :]
                           + d2[:, :, None] * wd[None, :, :]
                           + dist[:, :, None] * wt[None, :, :])
                m2 = _silu(_dot(m1.reshape(CI * N, H), w2) + b2)
                m3 = m2.reshape(CI, N, H) * em_ref[0, sl, :][:, :, None]
                agg = jnp.sum(m3, axis=1)
                u = _silu(_dot(hc, nwh) + _dot(agg, nwa) + nb1)
                u = _dot(u, nw2) + nb2
                hn[sl, :] = (hc + u) * nm_ref[0, sl, :]
                return 0

            jax.lax.fori_loop(0, NCH, gcl_chunk, 0)
            h_cur = 1 - h_cur

        hr = h_refs[h_cur]
        hf = hr[:]
        b = _dot(hf, cwc_ref[layer])
        xn = x_refs[1 - x_cur]
        wr = cwr_ref[layer]
        wd = cwd_ref[layer]
        wt = cwt_ref[layer]
        b1 = cb1_ref[layer]
        w2 = cw2_ref[layer]
        b2 = cb2_ref[layer]
        w3 = cw3_ref[layer]

        def coord_chunk(c, _, hf=hf, b=b, hr=hr, xr=xr, xn=xn, wr=wr,
                        wd=wd, wt=wt, b1=b1, w2=w2, b2=b2, w3=w3):
            sl = pl.ds(c * CI, CI)
            hc = hr[sl, :]
            xc = xr[sl, :]
            d2 = d2_ref[sl, :]
            dist = dist_ref[sl, :]
            a = _dot(hc, wr) + b1
            p1 = _silu(a[:, None, :] + b[None, :, :]
                       + d2[:, :, None] * wd[None, :, :]
                       + dist[:, :, None] * wt[None, :, :])
            p2 = _silu(_dot(p1.reshape(CI * N, H), w2) + b2)
            s = jnp.sum(p2.reshape(CI, N, H) * w3[None, :, :], axis=2)
            norm = jnp.sqrt(d2 + 1e-8)
            w = s * em_ref[0, sl, :] / (norm + 1.0)
            delta = xc * jnp.sum(w, axis=1, keepdims=True) - _dot(w, xr[:])
            xn[sl, :] = xc + delta
            return 0

        jax.lax.fori_loop(0, NCH, coord_chunk, 0)
        x_cur = 1 - x_cur

    h = h_refs[h_cur][:]
    h = _silu(_dot(h, ow0_ref[:]) + ob0_ref[:])
    h = _silu(_dot(h, ow1_ref[:]) + ob1_ref[:])
    hf_ref[0] = (_dot(h, ow2_ref[:]) + ob2_ref[:]) * nm
    vel_ref[0] = (x_refs[x_cur][:] - x0) * nm


def _stack(blocks, get):
    return jnp.stack([get(b) for b in blocks])


def kernel(t, xh, node_mask, edge_mask, params):
    nm = node_mask
    xhm = xh * nm
    x0 = xhm[:, :, :3]
    ht = jnp.broadcast_to(t[:, None, :], (BS, N, 1))
    hin = jnp.concatenate([xhm[:, :, 3:], ht], axis=-1)
    em = edge_mask.reshape(BS, N, N)

    gcls = [g for blk in params["blocks"] for g in blk["gcls"]]
    coords = [blk["coord_mlp"] for blk in params["blocks"]]

    def first_splits(layers, idx):
        ws = [l[idx]["w"] for l in layers]
        return (jnp.stack([w[:H] for w in ws]),
                jnp.stack([w[H:2 * H] for w in ws]),
                jnp.stack([w[2 * H:2 * H + 1] for w in ws]),
                jnp.stack([w[2 * H + 1:2 * H + 2] for w in ws]),
                jnp.stack([l[idx]["b"].reshape(1, H) for l in layers]))

    edge_mlps = [g["edge_mlp"] for g in gcls]
    gwr, gwc, gwd, gwt, gb1 = first_splits(edge_mlps, 0)
    gw2 = jnp.stack([e[1]["w"] for e in edge_mlps])
    gb2 = jnp.stack([e[1]["b"].reshape(1, H) for e in edge_mlps])
    nwh = jnp.stack([g["node_mlp"][0]["w"][:H] for g in gcls])
    nwa = jnp.stack([g["node_mlp"][0]["w"][H:] * 0.01 for g in gcls])
    nb1 = jnp.stack([g["node_mlp"][0]["b"].reshape(1, H) for g in gcls])
    nw2 = jnp.stack([g["node_mlp"][1]["w"] for g in gcls])
    nb2 = jnp.stack([g["node_mlp"][1]["b"].reshape(1, H) for g in gcls])

    gwr2, gwc2, gwd2, gwt2, cb1 = first_splits(coords, 0)
    cw2 = jnp.stack([c[1]["w"] for c in coords])
    cb2 = jnp.stack([c[1]["b"].reshape(1, H) for c in coords])
    cw3 = jnp.stack([c[2]["w"].reshape(1, H) * 0.01 for c in coords])

    emb = params["embedding"]
    eo = params["embedding_out"]

    full = lambda s: pl.BlockSpec(s, lambda b, _s=len(s): (0,) * _s)
    per_mol = lambda s: pl.BlockSpec(s, lambda b: (b,) + (0,) * (len(s) - 1))

    vel, hf = pl.pallas_call(
        _fused_kernel,
        grid=(BS,),
        in_specs=[
            per_mol((1, N, IN_NF)), per_mol((1, N, 3)),
            per_mol((1, N, N)), per_mol((1, N, 1)),
            full((IN_NF, H)), full((1, H)), full((H, H)), full((1, H)),
            full((SUB * N_LAYERS, H, H)), full((SUB * N_LAYERS, H, H)),
            full((SUB * N_LAYERS, 1, H)), full((SUB * N_LAYERS, 1, H)),
            full((SUB * N_LAYERS, 1, H)), full((SUB * N_LAYERS, H, H)),
            full((SUB * N_LAYERS, 1, H)),
            full((SUB * N_LAYERS, H, H)), full((SUB * N_LAYERS, H, H)),
            full((SUB * N_LAYERS, 1, H)), full((SUB * N_LAYERS, H, H)),
            full((SUB * N_LAYERS, 1, H)),
            full((N_LAYERS, H, H)), full((N_LAYERS, H, H)),
            full((N_LAYERS, 1, H)), full((N_LAYERS, 1, H)),
            full((N_LAYERS, 1, H)), full((N_LAYERS, H, H)),
            full((N_LAYERS, 1, H)), full((N_LAYERS, 1, H)),
            full((H, H)), full((1, H)), full((H, H)), full((1, H)),
            full((H, IN_NF)), full((1, IN_NF)),
        ],
        out_specs=[per_mol((1, N, 3)), per_mol((1, N, IN_NF))],
        out_shape=[jax.ShapeDtypeStruct((BS, N, 3), jnp.float32),
                   jax.ShapeDtypeStruct((BS, N, IN_NF), jnp.float32)],
        scratch_shapes=[
            pltpu.VMEM((N, H), jnp.float32), pltpu.VMEM((N, H), jnp.float32),
            pltpu.VMEM((N, 3), jnp.float32), pltpu.VMEM((N, 3), jnp.float32),
            pltpu.VMEM((N, N), jnp.float32), pltpu.VMEM((N, N), jnp.float32),
        ],
        compiler_params=pltpu.CompilerParams(
            dimension_semantics=("parallel",)),
    )(hin, x0, em, nm,
      emb[0]["w"], emb[0]["b"].reshape(1, H),
      emb[1]["w"], emb[1]["b"].reshape(1, H),
      gwr, gwc, gwd, gwt, gb1, gw2, gb2,
      nwh, nwa, nb1, nw2, nb2,
      gwr2, gwc2, gwd2, gwt2, cb1, cw2, cb2, cw3,
      eo[0]["w"], eo[0]["b"].reshape(1, H),
      eo[1]["w"], eo[1]["b"].reshape(1, H),
      eo[2]["w"], eo[2]["b"].reshape(1, IN_NF))

    return jnp.concatenate([vel, hf[:, :, :8]], axis=-1)


# speedup vs baseline: 16.2919x; 1.1601x over previous
import jax
import jax.numpy as jnp
from jax.experimental import pallas as pl
from jax.experimental.pallas import tpu as pltpu

BS = 16
N = 64
H = 128
IN_NF = 9
CI = 64
NCH = N // CI
N_LAYERS = 4
SUB = 2


def _silu(v):
    return v * jax.nn.sigmoid(v)


def _dot(a, b):
    return jnp.dot(a, b, preferred_element_type=jnp.float32)


def _nt(a, b):
    return jax.lax.dot_general(a, b, (((1,), (1,)), ((), ())),
                               preferred_element_type=jnp.float32)


def _pairwise_full(x):
    g = _nt(x, x)
    eye = (jax.lax.broadcasted_iota(jnp.int32, (N, N), 0) ==
           jax.lax.broadcasted_iota(jnp.int32, (N, N), 1)).astype(jnp.float32)
    rrow = jnp.sum(g * eye, axis=0, keepdims=True)
    rcol = jnp.sum(g * eye, axis=1, keepdims=True)
    return jnp.maximum(rcol + rrow - 2.0 * g, 0.0)


def _fused_kernel(hin_ref, x0_ref, em_ref, nm_ref,
                  ew0_ref, eb0_ref, ew1_ref, eb1e_ref,
                  gwr_ref, gwc_ref, gwd_ref, gwt_ref, gb1_ref,
                  gw2_ref, gb2_ref,
                  nwh_ref, nwa_ref, nb1_ref, nw2_ref, nb2_ref,
                  cwr_ref, cwc_ref, cwd_ref, cwt_ref, cb1_ref,
                  cw2_ref, cb2_ref, cw3_ref,
                  ow0_ref, ob0_ref, ow1_ref, ob1_ref, ow2_ref, ob2_ref,
                  vel_ref, hf_ref,
                  ha_ref, hb_ref, xa_ref, xb_ref, d2_ref, dist_ref):
    nm = nm_ref[0]
    x0 = x0_ref[0]

    h = _silu(_dot(hin_ref[0], ew0_ref[:]) + eb0_ref[:])
    ha_ref[:] = _dot(h, ew1_ref[:]) + eb1e_ref[:]

    dist_ref[:] = _pairwise_full(x0)

    h_refs = (ha_ref, hb_ref)
    x_refs = (xa_ref, xb_ref)
    h_cur = 0
    x_cur = 0
    xa_ref[:] = x0

    for layer in range(N_LAYERS):
        xr = x_refs[x_cur]
        d2_ref[:] = _pairwise_full(xr[:])

        for sub in range(SUB):
            k = layer * SUB + sub
            hr = h_refs[h_cur]
            hn = h_refs[1 - h_cur]
            hf = hr[:]
            b = _dot(hf, gwc_ref[k])
            wr = gwr_ref[k]
            wd = gwd_ref[k]
            wt = gwt_ref[k]
            b1 = gb1_ref[k]
            w2 = gw2_ref[k]
            b2 = gb2_ref[k]
            nwh = nwh_ref[k]
            nwa = nwa_ref[k]
            nb1 = nb1_ref[k]
            nw2 = nw2_ref[k]
            nb2 = nb2_ref[k]

            def gcl_chunk(c, _, hf=hf, b=b, hr=hr, hn=hn, wr=wr, wd=wd,
                          wt=wt, b1=b1, w2=w2, b2=b2, nwh=nwh, nwa=nwa,
                          nb1=nb1, nw2=nw2, nb2=nb2):
                sl = pl.ds(c * CI, CI)
                hc = hr[sl, :]
                d2 = d2_ref[sl, :]
                dist = dist_ref[sl, :]
                a = _dot(hc, wr) + b1
                m1 = _silu(a[:, None, :] + b[None, :, :]
                           + d2[:, :, None] * wd[None, :, :]
                           + dist[:, :, None] * wt[None, :, :])
                m2 = _silu(_dot(m1.reshape(CI * N, H), w2) + b2)
                m3 = m2.reshape(CI, N, H) * em_ref[0, sl, :][:, :, None]
                agg = jnp.sum(m3, axis=1)
                u = _silu(_dot(hc, nwh) + _dot(agg, nwa) + nb1)
                u = _dot(u, nw2) + nb2
                hn[sl, :] = (hc + u) * nm_ref[0, sl, :]
                return 0

            jax.lax.fori_loop(0, NCH, gcl_chunk, 0)
            h_cur = 1 - h_cur

        hr = h_refs[h_cur]
        hf = hr[:]
        b = _dot(hf, cwc_ref[layer])
        xn = x_refs[1 - x_cur]
        wr = cwr_ref[layer]
        wd = cwd_ref[layer]
        wt = cwt_ref[layer]
        b1 = cb1_ref[layer]
        w2 = cw2_ref[layer]
        b2 = cb2_ref[layer]
        w3 = cw3_ref[layer]

        def coord_chunk(c, _, hf=hf, b=b, hr=hr, xr=xr, xn=xn, wr=wr,
                        wd=wd, wt=wt, b1=b1, w2=w2, b2=b2, w3=w3):
            sl = pl.ds(c * CI, CI)
            hc = hr[sl, :]
            xc = xr[sl, :]
            d2 = d2_ref[sl, :]
            dist = dist_ref[sl, :]
            a = _dot(hc, wr) + b1
            p1 = _silu(a[:, None, :] + b[None, :, :]
                       + d2[:, :, None] * wd[None, :, :]
                       + dist[:, :, None] * wt[None, :, :])
            p2 = _silu(_dot(p1.reshape(CI * N, H), w2) + b2)
            s = jnp.sum(p2.reshape(CI, N, H) * w3[None, :, :], axis=2)
            norm = jnp.sqrt(d2 + 1e-8)
            w = s * em_ref[0, sl, :] / (norm + 1.0)
            delta = xc * jnp.sum(w, axis=1, keepdims=True) - _dot(w, xr[:])
            xn[sl, :] = xc + delta
            return 0

        jax.lax.fori_loop(0, NCH, coord_chunk, 0)
        x_cur = 1 - x_cur

    h = h_refs[h_cur][:]
    h = _silu(_dot(h, ow0_ref[:]) + ob0_ref[:])
    h = _silu(_dot(h, ow1_ref[:]) + ob1_ref[:])
    hf_ref[0] = (_dot(h, ow2_ref[:]) + ob2_ref[:]) * nm
    vel_ref[0] = (x_refs[x_cur][:] - x0) * nm


def _stack(blocks, get):
    return jnp.stack([get(b) for b in blocks])


def kernel(t, xh, node_mask, edge_mask, params):
    nm = node_mask
    xhm = xh * nm
    x0 = xhm[:, :, :3]
    ht = jnp.broadcast_to(t[:, None, :], (BS, N, 1))
    hin = jnp.concatenate([xhm[:, :, 3:], ht], axis=-1)
    em = edge_mask.reshape(BS, N, N)

    gcls = [g for blk in params["blocks"] for g in blk["gcls"]]
    coords = [blk["coord_mlp"] for blk in params["blocks"]]

    def first_splits(layers, idx):
        ws = [l[idx]["w"] for l in layers]
        return (jnp.stack([w[:H] for w in ws]),
                jnp.stack([w[H:2 * H] for w in ws]),
                jnp.stack([w[2 * H:2 * H + 1] for w in ws]),
                jnp.stack([w[2 * H + 1:2 * H + 2] for w in ws]),
                jnp.stack([l[idx]["b"].reshape(1, H) for l in layers]))

    edge_mlps = [g["edge_mlp"] for g in gcls]
    gwr, gwc, gwd, gwt, gb1 = first_splits(edge_mlps, 0)
    gw2 = jnp.stack([e[1]["w"] for e in edge_mlps])
    gb2 = jnp.stack([e[1]["b"].reshape(1, H) for e in edge_mlps])
    nwh = jnp.stack([g["node_mlp"][0]["w"][:H] for g in gcls])
    nwa = jnp.stack([g["node_mlp"][0]["w"][H:] * 0.01 for g in gcls])
    nb1 = jnp.stack([g["node_mlp"][0]["b"].reshape(1, H) for g in gcls])
    nw2 = jnp.stack([g["node_mlp"][1]["w"] for g in gcls])
    nb2 = jnp.stack([g["node_mlp"][1]["b"].reshape(1, H) for g in gcls])

    gwr2, gwc2, gwd2, gwt2, cb1 = first_splits(coords, 0)
    cw2 = jnp.stack([c[1]["w"] for c in coords])
    cb2 = jnp.stack([c[1]["b"].reshape(1, H) for c in coords])
    cw3 = jnp.stack([c[2]["w"].reshape(1, H) * 0.01 for c in coords])

    emb = params["embedding"]
    eo = params["embedding_out"]

    full = lambda s: pl.BlockSpec(s, lambda b, _s=len(s): (0,) * _s)
    per_mol = lambda s: pl.BlockSpec(s, lambda b: (b,) + (0,) * (len(s) - 1))

    vel, hf = pl.pallas_call(
        _fused_kernel,
        grid=(BS,),
        in_specs=[
            per_mol((1, N, IN_NF)), per_mol((1, N, 3)),
            per_mol((1, N, N)), per_mol((1, N, 1)),
            full((IN_NF, H)), full((1, H)), full((H, H)), full((1, H)),
            full((SUB * N_LAYERS, H, H)), full((SUB * N_LAYERS, H, H)),
            full((SUB * N_LAYERS, 1, H)), full((SUB * N_LAYERS, 1, H)),
            full((SUB * N_LAYERS, 1, H)), full((SUB * N_LAYERS, H, H)),
            full((SUB * N_LAYERS, 1, H)),
            full((SUB * N_LAYERS, H, H)), full((SUB * N_LAYERS, H, H)),
            full((SUB * N_LAYERS, 1, H)), full((SUB * N_LAYERS, H, H)),
            full((SUB * N_LAYERS, 1, H)),
            full((N_LAYERS, H, H)), full((N_LAYERS, H, H)),
            full((N_LAYERS, 1, H)), full((N_LAYERS, 1, H)),
            full((N_LAYERS, 1, H)), full((N_LAYERS, H, H)),
            full((N_LAYERS, 1, H)), full((N_LAYERS, 1, H)),
            full((H, H)), full((1, H)), full((H, H)), full((1, H)),
            full((H, IN_NF)), full((1, IN_NF)),
        ],
        out_specs=[per_mol((1, N, 3)), per_mol((1, N, IN_NF))],
        out_shape=[jax.ShapeDtypeStruct((BS, N, 3), jnp.float32),
                   jax.ShapeDtypeStruct((BS, N, IN_NF), jnp.float32)],
        scratch_shapes=[
            pltpu.VMEM((N, H), jnp.float32), pltpu.VMEM((N, H), jnp.float32),
            pltpu.VMEM((N, 3), jnp.float32), pltpu.VMEM((N, 3), jnp.float32),
            pltpu.VMEM((N, N), jnp.float32), pltpu.VMEM((N, N), jnp.float32),
        ],
        compiler_params=pltpu.CompilerParams(
            dimension_semantics=("parallel",)),
    )(hin, x0, em, nm,
      emb[0]["w"], emb[0]["b"].reshape(1, H),
      emb[1]["w"], emb[1]["b"].reshape(1, H),
      gwr, gwc, gwd, gwt, gb1, gw2, gb2,
      nwh, nwa, nb1, nw2, nb2,
      gwr2, gwc2, gwd2, gwt2, cb1, cw2, cb2, cw3,
      eo[0]["w"], eo[0]["b"].reshape(1, H),
      eo[1]["w"], eo[1]["b"].reshape(1, H),
      eo[2]["w"], eo[2]["b"].reshape(1, IN_NF))

    return jnp.concatenate([vel, hf[:, :, :8]], axis=-1)
